# Initial kernel scaffold; baseline (speedup 1.0000x reference)
import jax
import jax.numpy as jnp
from jax.experimental import pallas as pl


def kernel(audio, actor, lstm_params, dec_W0, dec_b0, conv_params, up_vals, spiral_idx, up_rows, up_cols):
    raise NotImplementedError("write your pallas kernel here")


# speedup vs baseline: 1.4182x; 1.4182x over previous
import functools

import jax
import jax.numpy as jnp
from jax import lax
from jax.experimental import pallas as pl
from jax.experimental.pallas import tpu as pltpu
from jax.experimental.pallas import tpu_sc as plsc

_LVL = [10000, 2500, 625, 157, 40]
S = 12
H = 64
T = 32
NC = 2
NS = 16
NW = NC * NS
CH = 8


def _pad_n(n):
    m = 8 * NW
    return (n + m - 1) // m * m


def _pad_to(arr, n):
    if arr.size == n:
        return arr
    return jnp.concatenate([arr, jnp.zeros((n - arr.size,), arr.dtype)])


def _lstm_dec_body(audio_ref, Wf0, Uf0, bf0, Wb0, Ub0, bb0,
                   Wfs, Ufs, bfs, Wbs, Ubs, bbs, Wd, bd,
                   out_ref, gf_scr, gb_scr, hn_scr):
    def sigm(x):
        return 1.0 / (1.0 + jnp.exp(-x))

    def run_layer(h_val, Wf, Uf, bf, Wb, Ub, bb):
        gf_scr[...] = lax.dot_general(
            h_val, Wf, (((1,), (1,)), ((), ())),
            preferred_element_type=jnp.float32) + bf
        gb_scr[...] = lax.dot_general(
            h_val, Wb, (((1,), (1,)), ((), ())),
            preferred_element_type=jnp.float32) + bb

        def step(tt, carry, g_scr, U, col0):
            h, c = carry
            g = g_scr[pl.ds(tt, 1), :] + lax.dot_general(
                h, U, (((1,), (1,)), ((), ())),
                preferred_element_type=jnp.float32)
            i = sigm(g[:, 0:H])
            f = sigm(g[:, H:2 * H])
            gg = jnp.tanh(g[:, 2 * H:3 * H])
            o = sigm(g[:, 3 * H:4 * H])
            c = f * c + i * gg
            h = o * jnp.tanh(c)
            hn_scr[pl.ds(tt, 1), pl.ds(col0, H)] = h
            return (h, c)

        z = jnp.zeros((1, H), jnp.float32)
        lax.fori_loop(0, T, lambda t, cr: step(t, cr, gf_scr, Uf, 0), (z, z))
        lax.fori_loop(0, T, lambda t, cr: step(T - 1 - t, cr, gb_scr, Ub, H),
                      (z, z))
        return hn_scr[...]

    h = run_layer(audio_ref[...], Wf0[...], Uf0[...], bf0[...],
                  Wb0[...], Ub0[...], bb0[...])
    for i in range(4):
        h = run_layer(h, Wfs[i], Ufs[i], bfs[i], Wbs[i], Ubs[i], bbs[i])
    r = lax.dot_general(h, Wd[...], (((1,), (2,)), ((), ())),
                        preferred_element_type=jnp.float32)
    out_ref[...] = jnp.swapaxes(r, 0, 1) + bd[...]


def _lstm_dec(audio, l0, lrest, Wd, bd):
    return pl.pallas_call(
        _lstm_dec_body,
        out_shape=jax.ShapeDtypeStruct((40, T, 32), jnp.float32),
        scratch_shapes=[
            pltpu.VMEM((T, 4 * H), jnp.float32),
            pltpu.VMEM((T, 4 * H), jnp.float32),
            pltpu.VMEM((T, 2 * H), jnp.float32),
        ],
    )(audio, *l0, *lrest, Wd, bd)


def _pmm_body(y_ref, w_ref, b_ref, out_ref):
    n, t, c = y_ref.shape
    co = w_ref.shape[2]
    y2 = y_ref[...].reshape(n * t, c)
    r = lax.dot_general(y2, w_ref[0], (((1,), (0,)), ((), ())),
                        preferred_element_type=jnp.float32)
    r = r + b_ref[0]
    out_ref[...] = r.reshape(1, n, t, co)


def _pmm(y, wt, bb, Np, C, Co):
    Bn = 256
    NB = Np // Bn
    return pl.pallas_call(
        _pmm_body,
        grid=(NB, S),
        in_specs=[
            pl.BlockSpec((Bn, T, C), lambda nb, s: (nb, 0, 0)),
            pl.BlockSpec((1, C, Co), lambda nb, s: (s, 0, 0)),
            pl.BlockSpec((1, 1, Co), lambda nb, s: (0, 0, 0)),
        ],
        out_specs=pl.BlockSpec((1, Bn, T, Co), lambda nb, s: (s, nb, 0, 0)),
        out_shape=jax.ShapeDtypeStruct((S, Np, T, Co), jnp.float32),
    )(y, wt, bb)


def _make_pool(Np, D):
    bn = Np // NW
    mesh = plsc.VectorSubcoreMesh(core_axis_name="c", subcore_axis_name="s")

    @functools.partial(
        pl.kernel, mesh=mesh,
        out_type=jax.ShapeDtypeStruct((Np, D), jnp.float32),
        scratch_types=[
            pltpu.VMEM((CH * 4,), jnp.int32),
            pltpu.VMEM((CH * 4,), jnp.float32),
            pltpu.VMEM((CH * 4, D), jnp.float32),
            pltpu.VMEM((CH, D), jnp.float32),
            pltpu.SemaphoreType.DMA,
        ],
    )
    def pool_k(src_hbm, cols_hbm, vals_hbm, out_hbm, idx_v, w_v, g_v, y_v, sem):
        wid = lax.axis_index("s") * NC + lax.axis_index("c")
        base = wid * bn

        def chunk(ci, carry):
            n0 = base + ci * CH
            pltpu.sync_copy(cols_hbm.at[pl.ds(n0 * 4, CH * 4)], idx_v)
            pltpu.sync_copy(vals_hbm.at[pl.ds(n0 * 4, CH * 4)], w_v)
            pltpu.async_copy(src_hbm.at[idx_v], g_v, sem).wait()
            wvecs = [w_v[pl.ds(16 * j, 16)] for j in range((CH * 4) // 16)]
            for i in range(CH):
                wv = wvecs[(4 * i) // 16]
                o = (4 * i) % 16
                w0 = wv[o]
                w1 = wv[o + 1]
                w2 = wv[o + 2]
                w3 = wv[o + 3]

                def vec(dv, c2):
                    sl = pl.ds(dv * 16, 16)
                    y_v[i, sl] = (g_v[4 * i, sl] * w0 + g_v[4 * i + 1, sl] * w1
                                  + g_v[4 * i + 2, sl] * w2
                                  + g_v[4 * i + 3, sl] * w3)
                    return c2

                lax.fori_loop(0, D // 16, vec, 0)
            pltpu.sync_copy(y_v, out_hbm.at[pl.ds(n0, CH)])
            return carry

        lax.fori_loop(0, bn // CH, chunk, 0)

    return pool_k


def _make_gsum(Np, D, elu):
    bn = Np // NW
    mesh = plsc.VectorSubcoreMesh(core_axis_name="c", subcore_axis_name="s")

    @functools.partial(
        pl.kernel, mesh=mesh,
        out_type=jax.ShapeDtypeStruct((Np, D), jnp.float32),
        scratch_types=[
            pltpu.VMEM((CH * S,), jnp.int32),
            pltpu.VMEM((CH * S, D), jnp.float32),
            pltpu.VMEM((CH, D), jnp.float32),
            pltpu.SemaphoreType.DMA,
        ],
    )
    def gsum_k(p_hbm, gidx_hbm, out_hbm, idx_v, g_v, y_v, sem):
        wid = lax.axis_index("s") * NC + lax.axis_index("c")
        base = wid * bn

        def chunk(ci, carry):
            n0 = base + ci * CH
            pltpu.sync_copy(gidx_hbm.at[pl.ds(n0 * S, CH * S)], idx_v)
            pltpu.async_copy(p_hbm.at[idx_v], g_v, sem).wait()
            for i in range(CH):
                def vec(dv, c2):
                    sl = pl.ds(dv * 16, 16)
                    v = g_v[S * i, sl]
                    for s in range(1, S):
                        v = v + g_v[S * i + s, sl]
                    if elu:
                        v = jnp.where(v > 0.0, v,
                                      jnp.exp(jnp.minimum(v, 0.0)) - 1.0)
                    y_v[i, sl] = v
                    return c2

                lax.fori_loop(0, D // 16, vec, 0)
            pltpu.sync_copy(y_v, out_hbm.at[pl.ds(n0, CH)])
            return carry

        lax.fori_loop(0, bn // CH, chunk, 0)

    return gsum_k


def kernel(audio, actor, lstm_params, dec_W0, dec_b0, conv_params, up_vals,
           spiral_idx, up_rows, up_cols):
    (Wf0, Uf0, bf10, bf20, Wb0, Ub0, bb10, bb20) = lstm_params[0]
    l0 = (Wf0, Uf0, (bf10 + bf20).reshape(1, -1),
          Wb0, Ub0, (bb10 + bb20).reshape(1, -1))
    Wfs = jnp.stack([lstm_params[i][0] for i in range(1, 5)])
    Ufs = jnp.stack([lstm_params[i][1] for i in range(1, 5)])
    bfs = jnp.stack([(lstm_params[i][2] + lstm_params[i][3]).reshape(1, -1)
                     for i in range(1, 5)])
    Wbs = jnp.stack([lstm_params[i][4] for i in range(1, 5)])
    Ubs = jnp.stack([lstm_params[i][5] for i in range(1, 5)])
    bbs = jnp.stack([(lstm_params[i][6] + lstm_params[i][7]).reshape(1, -1)
                     for i in range(1, 5)])
    lrest = (Wfs, Ufs, bfs, Wbs, Ubs, bbs)
    Wd = dec_W0.reshape(40, 32, 2 * H)
    bd = dec_b0.reshape(40, 1, 32)

    x = _lstm_dec(audio, l0, lrest, Wd, bd)
    src = x.reshape(40, T * 32)
    C = 32
    Np = 40
    for k in range(5):
        if k < 4:
            lvl = 3 - k
            Nd = _LVL[lvl]
            Np = _pad_n(Nd)
            cols = _pad_to(up_cols[lvl], Np * 4)
            vals = _pad_to(up_vals[lvl], Np * 4)
            y = _make_pool(Np, T * C)(src, cols, vals)
            sidx = spiral_idx[lvl]
        else:
            y = src
            sidx = spiral_idx[0]
        W, b = conv_params[k]
        if k == 4:
            W = jnp.concatenate([W, jnp.zeros((1, W.shape[1]), W.dtype)])
            b = jnp.concatenate([b, jnp.zeros((1,), b.dtype)])
        Co = W.shape[0]
        wt = jnp.transpose(W.reshape(Co, S, C), (1, 2, 0))
        bb = (b / S).reshape(1, 1, Co)
        P = _pmm(y.reshape(Np, T, C), wt, bb, Np, C, Co)
        gidx = sidx + (jnp.arange(S, dtype=jnp.int32) * Np)[None, :]
        gidx = _pad_to(gidx.reshape(-1), Np * S)
        src = _make_gsum(Np, T * Co, k < 4)(
            P.reshape(S * Np, T * Co), gidx)
        C = Co

    pred = src[:_LVL[0]].reshape(_LVL[0], T, 4)[:, :, :3].transpose(1, 0, 2)
    return pred + actor


# speedup vs baseline: 3.6371x; 2.5645x over previous
import functools

import jax
import jax.numpy as jnp
from jax import lax
from jax.experimental import pallas as pl
from jax.experimental.pallas import tpu as pltpu
from jax.experimental.pallas import tpu_sc as plsc

_LVL = [10000, 2500, 625, 157, 40]
S = 12
H = 64
T = 32
NC = 2
NS = 16
NW = NC * NS
CH = 8


def _pad_n(n):
    m = 8 * NW
    return (n + m - 1) // m * m


def _pad_to(arr, n):
    if arr.size == n:
        return arr
    return jnp.concatenate([arr, jnp.zeros((n - arr.size,), arr.dtype)])


def _lstm_dec_body(audio_ref, Wf0, Uf0, bf0, Wb0, Ub0, bb0,
                   Wfs, Ufs, bfs, Wbs, Ubs, bbs, Wd, bd,
                   out_ref, gf_scr, gb_scr, hn_scr):
    def sigm(x):
        return 1.0 / (1.0 + jnp.exp(-x))

    def run_layer(h_val, Wf, Uf, bf, Wb, Ub, bb):
        gf_scr[...] = lax.dot_general(
            h_val, Wf, (((1,), (1,)), ((), ())),
            preferred_element_type=jnp.float32) + bf
        gb_scr[...] = lax.dot_general(
            h_val, Wb, (((1,), (1,)), ((), ())),
            preferred_element_type=jnp.float32) + bb

        def step(tt, carry, g_scr, U, col0):
            h, c = carry
            g = g_scr[pl.ds(tt, 1), :] + lax.dot_general(
                h, U, (((1,), (1,)), ((), ())),
                preferred_element_type=jnp.float32)
            i = sigm(g[:, 0:H])
            f = sigm(g[:, H:2 * H])
            gg = jnp.tanh(g[:, 2 * H:3 * H])
            o = sigm(g[:, 3 * H:4 * H])
            c = f * c + i * gg
            h = o * jnp.tanh(c)
            hn_scr[pl.ds(tt, 1), pl.ds(col0, H)] = h
            return (h, c)

        z = jnp.zeros((1, H), jnp.float32)
        lax.fori_loop(0, T, lambda t, cr: step(t, cr, gf_scr, Uf, 0), (z, z))
        lax.fori_loop(0, T, lambda t, cr: step(T - 1 - t, cr, gb_scr, Ub, H),
                      (z, z))
        return hn_scr[...]

    h = run_layer(audio_ref[...], Wf0[...], Uf0[...], bf0[...],
                  Wb0[...], Ub0[...], bb0[...])
    for i in range(4):
        h = run_layer(h, Wfs[i], Ufs[i], bfs[i], Wbs[i], Ubs[i], bbs[i])
    r = lax.dot_general(h, Wd[...], (((1,), (2,)), ((), ())),
                        preferred_element_type=jnp.float32)
    out_ref[...] = jnp.swapaxes(r, 0, 1) + bd[...]


def _lstm_dec(audio, l0, lrest, Wd, bd):
    return pl.pallas_call(
        _lstm_dec_body,
        out_shape=jax.ShapeDtypeStruct((40, T, 32), jnp.float32),
        scratch_shapes=[
            pltpu.VMEM((T, 4 * H), jnp.float32),
            pltpu.VMEM((T, 4 * H), jnp.float32),
            pltpu.VMEM((T, 2 * H), jnp.float32),
        ],
    )(audio, *l0, *lrest, Wd, bd)


def _pmm_body(y_ref, w_ref, b_ref, out_ref):
    r = lax.dot_general(y_ref[...], w_ref[0], (((1,), (0,)), ((), ())),
                        preferred_element_type=jnp.float32)
    out_ref[...] = r + b_ref[...]


def _pmm(y, wbd, bt, Np, D1, D2):
    Bn = 256
    NB = Np // Bn
    return pl.pallas_call(
        _pmm_body,
        grid=(NB, S),
        in_specs=[
            pl.BlockSpec((Bn, D1), lambda nb, s: (nb, 0)),
            pl.BlockSpec((1, D1, D2), lambda nb, s: (s, 0, 0)),
            pl.BlockSpec((1, D2), lambda nb, s: (0, 0)),
        ],
        out_specs=pl.BlockSpec((Bn, D2), lambda nb, s: (s * NB + nb, 0)),
        out_shape=jax.ShapeDtypeStruct((S * Np, D2), jnp.float32),
    )(y, wbd, bt)


def _make_pool(Np, D):
    bn = Np // NW
    mesh = plsc.VectorSubcoreMesh(core_axis_name="c", subcore_axis_name="s")

    @functools.partial(
        pl.kernel, mesh=mesh,
        out_type=jax.ShapeDtypeStruct((Np, D), jnp.float32),
        scratch_types=[
            pltpu.VMEM((CH * 4,), jnp.int32),
            pltpu.VMEM((CH * 4,), jnp.float32),
            pltpu.VMEM((CH * 4, D), jnp.float32),
            pltpu.VMEM((CH, D), jnp.float32),
            pltpu.SemaphoreType.DMA,
        ],
    )
    def pool_k(src_hbm, cols_hbm, vals_hbm, out_hbm, idx_v, w_v, g_v, y_v, sem):
        wid = lax.axis_index("s") * NC + lax.axis_index("c")
        base = wid * bn

        def chunk(ci, carry):
            n0 = base + ci * CH
            pltpu.sync_copy(cols_hbm.at[pl.ds(n0 * 4, CH * 4)], idx_v)
            pltpu.sync_copy(vals_hbm.at[pl.ds(n0 * 4, CH * 4)], w_v)
            pltpu.async_copy(src_hbm.at[idx_v], g_v, sem).wait()
            wvecs = [w_v[pl.ds(16 * j, 16)] for j in range((CH * 4) // 16)]
            for i in range(CH):
                wv = wvecs[(4 * i) // 16]
                o = (4 * i) % 16
                w0 = wv[o]
                w1 = wv[o + 1]
                w2 = wv[o + 2]
                w3 = wv[o + 3]

                def vec(dv, c2):
                    sl = pl.ds(dv * 16, 16)
                    y_v[i, sl] = (g_v[4 * i, sl] * w0 + g_v[4 * i + 1, sl] * w1
                                  + g_v[4 * i + 2, sl] * w2
                                  + g_v[4 * i + 3, sl] * w3)
                    return c2

                lax.fori_loop(0, D // 16, vec, 0)
            pltpu.sync_copy(y_v, out_hbm.at[pl.ds(n0, CH)])
            return carry

        lax.fori_loop(0, bn // CH, chunk, 0)

    return pool_k


def _make_gsum(Np, D, elu):
    bn = Np // NW
    mesh = plsc.VectorSubcoreMesh(core_axis_name="c", subcore_axis_name="s")

    @functools.partial(
        pl.kernel, mesh=mesh,
        out_type=jax.ShapeDtypeStruct((Np, D), jnp.float32),
        scratch_types=[
            pltpu.VMEM((CH * S,), jnp.int32),
            pltpu.VMEM((CH * S, D), jnp.float32),
            pltpu.VMEM((CH, D), jnp.float32),
            pltpu.SemaphoreType.DMA,
        ],
    )
    def gsum_k(p_hbm, gidx_hbm, out_hbm, idx_v, g_v, y_v, sem):
        wid = lax.axis_index("s") * NC + lax.axis_index("c")
        base = wid * bn

        def chunk(ci, carry):
            n0 = base + ci * CH
            pltpu.sync_copy(gidx_hbm.at[pl.ds(n0 * S, CH * S)], idx_v)
            pltpu.async_copy(p_hbm.at[idx_v], g_v, sem).wait()
            for i in range(CH):
                def vec(dv, c2):
                    sl = pl.ds(dv * 16, 16)
                    v = g_v[S * i, sl]
                    for s in range(1, S):
                        v = v + g_v[S * i + s, sl]
                    if elu:
                        v = jnp.where(v > 0.0, v,
                                      jnp.exp(jnp.minimum(v, 0.0)) - 1.0)
                    y_v[i, sl] = v
                    return c2

                lax.fori_loop(0, D // 16, vec, 0)
            pltpu.sync_copy(y_v, out_hbm.at[pl.ds(n0, CH)])
            return carry

        lax.fori_loop(0, bn // CH, chunk, 0)

    return gsum_k


def kernel(audio, actor, lstm_params, dec_W0, dec_b0, conv_params, up_vals,
           spiral_idx, up_rows, up_cols):
    (Wf0, Uf0, bf10, bf20, Wb0, Ub0, bb10, bb20) = lstm_params[0]
    l0 = (Wf0, Uf0, (bf10 + bf20).reshape(1, -1),
          Wb0, Ub0, (bb10 + bb20).reshape(1, -1))
    Wfs = jnp.stack([lstm_params[i][0] for i in range(1, 5)])
    Ufs = jnp.stack([lstm_params[i][1] for i in range(1, 5)])
    bfs = jnp.stack([(lstm_params[i][2] + lstm_params[i][3]).reshape(1, -1)
                     for i in range(1, 5)])
    Wbs = jnp.stack([lstm_params[i][4] for i in range(1, 5)])
    Ubs = jnp.stack([lstm_params[i][5] for i in range(1, 5)])
    bbs = jnp.stack([(lstm_params[i][6] + lstm_params[i][7]).reshape(1, -1)
                     for i in range(1, 5)])
    lrest = (Wfs, Ufs, bfs, Wbs, Ubs, bbs)
    Wd = dec_W0.reshape(40, 32, 2 * H)
    bd = dec_b0.reshape(40, 1, 32)

    x = _lstm_dec(audio, l0, lrest, Wd, bd)
    src = x.reshape(40, T * 32)
    C = 32
    Np = 40
    for k in range(5):
        if k < 4:
            lvl = 3 - k
            Nd = _LVL[lvl]
            Np = _pad_n(Nd)
            cols = _pad_to(up_cols[lvl], Np * 4)
            vals = _pad_to(up_vals[lvl], Np * 4)
            y = _make_pool(Np, T * C)(src, cols, vals)
            sidx = spiral_idx[lvl]
        else:
            y = src
            sidx = spiral_idx[0]
        W, b = conv_params[k]
        if k == 4:
            W = jnp.concatenate([W, jnp.zeros((1, W.shape[1]), W.dtype)])
            b = jnp.concatenate([b, jnp.zeros((1,), b.dtype)])
        Co = W.shape[0]
        wt = jnp.transpose(W.reshape(Co, S, C), (1, 2, 0))
        wbd = jnp.einsum('tu,sco->stcuo', jnp.eye(T, dtype=jnp.float32),
                         wt).reshape(S, T * C, T * Co)
        bt = jnp.tile(b / S, (T,)).reshape(1, T * Co)
        P = _pmm(y, wbd, bt, Np, T * C, T * Co)
        gidx = sidx + (jnp.arange(S, dtype=jnp.int32) * Np)[None, :]
        gidx = _pad_to(gidx.reshape(-1), Np * S)
        src = _make_gsum(Np, T * Co, k < 4)(P, gidx)
        C = Co

    pred = src[:_LVL[0]].reshape(_LVL[0], T, 4)[:, :, :3].transpose(1, 0, 2)
    return pred + actor


# speedup vs baseline: 4.2657x; 1.1728x over previous
import functools

import jax
import jax.numpy as jnp
from jax import lax
from jax.experimental import pallas as pl
from jax.experimental.pallas import tpu as pltpu
from jax.experimental.pallas import tpu_sc as plsc

_LVL = [10000, 2500, 625, 157, 40]
S = 12
H = 64
T = 32
NC = 2
NS = 16
NW = NC * NS
CH = 8


def _pad_n(n):
    m = 8 * NW
    return (n + m - 1) // m * m


def _pad_to(arr, n):
    if arr.size == n:
        return arr
    return jnp.concatenate([arr, jnp.zeros((n - arr.size,), arr.dtype)])


def _lstm_dec_body(audio_ref, Wf0, Uf0, bf0, Wb0, Ub0, bb0,
                   Wfs, Ufs, bfs, Wbs, Ubs, bbs, Wd, bd,
                   out_ref, gf_scr, gb_scr, hn_scr):
    def sigm(x):
        return 1.0 / (1.0 + jnp.exp(-x))

    def run_layer(h_val, Wf, Uf, bf, Wb, Ub, bb):
        gf_scr[...] = lax.dot_general(
            h_val, Wf, (((1,), (1,)), ((), ())),
            preferred_element_type=jnp.float32) + bf
        gb_scr[...] = lax.dot_general(
            h_val, Wb, (((1,), (1,)), ((), ())),
            preferred_element_type=jnp.float32) + bb

        def step(tt, carry, g_scr, U, col0):
            h, c = carry
            g = g_scr[pl.ds(tt, 1), :] + lax.dot_general(
                h, U, (((1,), (1,)), ((), ())),
                preferred_element_type=jnp.float32)
            i = sigm(g[:, 0:H])
            f = sigm(g[:, H:2 * H])
            gg = jnp.tanh(g[:, 2 * H:3 * H])
            o = sigm(g[:, 3 * H:4 * H])
            c = f * c + i * gg
            h = o * jnp.tanh(c)
            hn_scr[pl.ds(tt, 1), pl.ds(col0, H)] = h
            return (h, c)

        z = jnp.zeros((1, H), jnp.float32)
        lax.fori_loop(0, T, lambda t, cr: step(t, cr, gf_scr, Uf, 0), (z, z))
        lax.fori_loop(0, T, lambda t, cr: step(T - 1 - t, cr, gb_scr, Ub, H),
                      (z, z))
        return hn_scr[...]

    h = run_layer(audio_ref[...], Wf0[...], Uf0[...], bf0[...],
                  Wb0[...], Ub0[...], bb0[...])
    for i in range(4):
        h = run_layer(h, Wfs[i], Ufs[i], bfs[i], Wbs[i], Ubs[i], bbs[i])
    r = lax.dot_general(h, Wd[...], (((1,), (2,)), ((), ())),
                        preferred_element_type=jnp.float32)
    out_ref[...] = jnp.swapaxes(r, 0, 1) + bd[...]


def _lstm_dec(audio, l0, lrest, Wd, bd):
    return pl.pallas_call(
        _lstm_dec_body,
        out_shape=jax.ShapeDtypeStruct((40, T, 32), jnp.float32),
        scratch_shapes=[
            pltpu.VMEM((T, 4 * H), jnp.float32),
            pltpu.VMEM((T, 4 * H), jnp.float32),
            pltpu.VMEM((T, 2 * H), jnp.float32),
        ],
    )(audio, *l0, *lrest, Wd, bd)


def _pmm(y, wbd, bt, Np, D1, D2):
    Bn = 256
    NB = Np // Bn
    w_res = S * D1 * D2 * 4 <= 16 * 1024 * 1024

    def body(y_ref, w_ref, b_ref, out_ref):
        w = w_ref[pl.program_id(1)] if w_res else w_ref[0]
        r = lax.dot_general(y_ref[...], w, (((1,), (0,)), ((), ())),
                            preferred_element_type=jnp.float32)
        out_ref[...] = r + b_ref[...]

    if w_res:
        w_spec = pl.BlockSpec((S, D1, D2), lambda nb, s: (0, 0, 0))
    else:
        w_spec = pl.BlockSpec((1, D1, D2), lambda nb, s: (s, 0, 0))
    return pl.pallas_call(
        body,
        grid=(NB, S),
        in_specs=[
            pl.BlockSpec((Bn, D1), lambda nb, s: (nb, 0)),
            w_spec,
            pl.BlockSpec((1, D2), lambda nb, s: (0, 0)),
        ],
        out_specs=pl.BlockSpec((Bn, D2), lambda nb, s: (s * NB + nb, 0)),
        out_shape=jax.ShapeDtypeStruct((S * Np, D2), jnp.float32),
    )(y, wbd, bt)


def _make_pool(Np, D):
    bn = Np // NW
    ch = CH
    nch = bn // ch
    mesh = plsc.VectorSubcoreMesh(core_axis_name="c", subcore_axis_name="s")

    @functools.partial(
        pl.kernel, mesh=mesh,
        out_type=jax.ShapeDtypeStruct((Np, D), jnp.float32),
        scratch_types=[
            pltpu.VMEM((2, ch * 4), jnp.int32),
            pltpu.VMEM((2, ch * 4), jnp.float32),
            pltpu.VMEM((2, ch * 4, D), jnp.float32),
            pltpu.VMEM((ch, D), jnp.float32),
            pltpu.SemaphoreType.DMA,
            pltpu.SemaphoreType.DMA,
        ],
    )
    def pool_k(src_hbm, cols_hbm, vals_hbm, out_hbm, idx_v, w_v, g_v, y_v,
               sem0, sem1):
        wid = lax.axis_index("s") * NC + lax.axis_index("c")
        base = wid * bn
        sems = (sem0, sem1)

        def fetch(n0, b):
            pltpu.sync_copy(cols_hbm.at[pl.ds(n0 * 4, ch * 4)], idx_v.at[b])
            pltpu.sync_copy(vals_hbm.at[pl.ds(n0 * 4, ch * 4)], w_v.at[b])
            pltpu.async_copy(src_hbm.at[idx_v.at[b]], g_v.at[b], sems[b])

        def drain(b):
            pltpu.make_async_copy(src_hbm.at[idx_v.at[b]], g_v.at[b],
                                  sems[b]).wait()

        def compute(n0, b):
            wvecs = [w_v[b, pl.ds(16 * j, 16)] for j in range((ch * 4) // 16)]
            for i in range(ch):
                wv = wvecs[(4 * i) // 16]
                o = (4 * i) % 16
                w0 = wv[o]
                w1 = wv[o + 1]
                w2 = wv[o + 2]
                w3 = wv[o + 3]

                def vec(dv, c2):
                    sl = pl.ds(dv * 16, 16)
                    y_v[i, sl] = (g_v[b, 4 * i, sl] * w0
                                  + g_v[b, 4 * i + 1, sl] * w1
                                  + g_v[b, 4 * i + 2, sl] * w2
                                  + g_v[b, 4 * i + 3, sl] * w3)
                    return c2

                lax.fori_loop(0, D // 16, vec, 0, unroll=4)
            pltpu.sync_copy(y_v, out_hbm.at[pl.ds(n0, ch)])

        if nch % 2 == 0:
            fetch(base, 0)

            def pair(ci, carry):
                n0 = base + (2 * ci) * ch
                fetch(n0 + ch, 1)
                drain(0)
                compute(n0, 0)

                @pl.when(ci + 1 < nch // 2)
                def _():
                    fetch(n0 + 2 * ch, 0)

                drain(1)
                compute(n0 + ch, 1)
                return carry

            lax.fori_loop(0, nch // 2, pair, 0)
        else:
            def chunk(ci, carry):
                n0 = base + ci * ch
                fetch(n0, 0)
                drain(0)
                compute(n0, 0)
                return carry

            lax.fori_loop(0, nch, chunk, 0)

    return pool_k


def _make_gsum(Np, D, elu):
    bn = Np // NW
    ch = 4 if D > 512 else 8
    nch = bn // ch
    mesh = plsc.VectorSubcoreMesh(core_axis_name="c", subcore_axis_name="s")

    @functools.partial(
        pl.kernel, mesh=mesh,
        out_type=jax.ShapeDtypeStruct((Np, D), jnp.float32),
        scratch_types=[
            pltpu.VMEM((2, ch * S), jnp.int32),
            pltpu.VMEM((2, ch * S, D), jnp.float32),
            pltpu.VMEM((ch, D), jnp.float32),
            pltpu.SemaphoreType.DMA,
            pltpu.SemaphoreType.DMA,
        ],
    )
    def gsum_k(p_hbm, gidx_hbm, out_hbm, idx_v, g_v, y_v, sem0, sem1):
        wid = lax.axis_index("s") * NC + lax.axis_index("c")
        base = wid * bn
        sems = (sem0, sem1)

        def fetch(n0, b):
            pltpu.sync_copy(gidx_hbm.at[pl.ds(n0 * S, ch * S)], idx_v.at[b])
            pltpu.async_copy(p_hbm.at[idx_v.at[b]], g_v.at[b], sems[b])

        def drain(b):
            pltpu.make_async_copy(p_hbm.at[idx_v.at[b]], g_v.at[b],
                                  sems[b]).wait()

        def compute(n0, b):
            for i in range(ch):
                def vec(dv, c2):
                    sl = pl.ds(dv * 16, 16)
                    v = g_v[b, S * i, sl]
                    for s in range(1, S):
                        v = v + g_v[b, S * i + s, sl]
                    if elu:
                        v = jnp.where(v > 0.0, v,
                                      jnp.exp(jnp.minimum(v, 0.0)) - 1.0)
                    y_v[i, sl] = v
                    return c2

                lax.fori_loop(0, D // 16, vec, 0, unroll=4)
            pltpu.sync_copy(y_v, out_hbm.at[pl.ds(n0, ch)])

        if nch % 2 == 0:
            fetch(base, 0)

            def pair(ci, carry):
                n0 = base + (2 * ci) * ch
                fetch(n0 + ch, 1)
                drain(0)
                compute(n0, 0)

                @pl.when(ci + 1 < nch // 2)
                def _():
                    fetch(n0 + 2 * ch, 0)

                drain(1)
                compute(n0 + ch, 1)
                return carry

            lax.fori_loop(0, nch // 2, pair, 0)
        else:
            def chunk(ci, carry):
                n0 = base + ci * ch
                fetch(n0, 0)
                drain(0)
                compute(n0, 0)
                return carry

            lax.fori_loop(0, nch, chunk, 0)

    return gsum_k


def kernel(audio, actor, lstm_params, dec_W0, dec_b0, conv_params, up_vals,
           spiral_idx, up_rows, up_cols):
    (Wf0, Uf0, bf10, bf20, Wb0, Ub0, bb10, bb20) = lstm_params[0]
    l0 = (Wf0, Uf0, (bf10 + bf20).reshape(1, -1),
          Wb0, Ub0, (bb10 + bb20).reshape(1, -1))
    Wfs = jnp.stack([lstm_params[i][0] for i in range(1, 5)])
    Ufs = jnp.stack([lstm_params[i][1] for i in range(1, 5)])
    bfs = jnp.stack([(lstm_params[i][2] + lstm_params[i][3]).reshape(1, -1)
                     for i in range(1, 5)])
    Wbs = jnp.stack([lstm_params[i][4] for i in range(1, 5)])
    Ubs = jnp.stack([lstm_params[i][5] for i in range(1, 5)])
    bbs = jnp.stack([(lstm_params[i][6] + lstm_params[i][7]).reshape(1, -1)
                     for i in range(1, 5)])
    lrest = (Wfs, Ufs, bfs, Wbs, Ubs, bbs)
    Wd = dec_W0.reshape(40, 32, 2 * H)
    bd = dec_b0.reshape(40, 1, 32)

    x = _lstm_dec(audio, l0, lrest, Wd, bd)
    src = x.reshape(40, T * 32)
    C = 32
    Np = 40
    for k in range(5):
        if k < 4:
            lvl = 3 - k
            Nd = _LVL[lvl]
            Np = _pad_n(Nd)
            cols = _pad_to(up_cols[lvl], Np * 4)
            vals = _pad_to(up_vals[lvl], Np * 4)
            y = _make_pool(Np, T * C)(src, cols, vals)
            sidx = spiral_idx[lvl]
        else:
            y = src
            sidx = spiral_idx[0]
        W, b = conv_params[k]
        if k == 4:
            W = jnp.concatenate([W, jnp.zeros((1, W.shape[1]), W.dtype)])
            b = jnp.concatenate([b, jnp.zeros((1,), b.dtype)])
        Co = W.shape[0]
        wt = jnp.transpose(W.reshape(Co, S, C), (1, 2, 0))
        wbd = jnp.zeros((S, T * C, T * Co), jnp.float32)
        for t in range(T):
            wbd = lax.dynamic_update_slice(wbd, wt, (0, t * C, t * Co))
        bt = jnp.tile(b / S, (T,)).reshape(1, T * Co)
        P = _pmm(y, wbd, bt, Np, T * C, T * Co)
        gidx = sidx + (jnp.arange(S, dtype=jnp.int32) * Np)[None, :]
        gidx = _pad_to(gidx.reshape(-1), Np * S)
        src = _make_gsum(Np, T * Co, k < 4)(P, gidx)
        C = Co

    pred = src[:_LVL[0]].reshape(_LVL[0], T, 4)[:, :, :3].transpose(1, 0, 2)
    return pred + actor


# speedup vs baseline: 4.3234x; 1.0135x over previous
import functools

import jax
import jax.numpy as jnp
from jax import lax
from jax.experimental import pallas as pl
from jax.experimental.pallas import tpu as pltpu
from jax.experimental.pallas import tpu_sc as plsc

_LVL = [10000, 2500, 625, 157, 40]
S = 12
H = 64
T = 32
NC = 2
NS = 16
NW = NC * NS
CH = 8


def _pad_n(n):
    m = 8 * NW
    return (n + m - 1) // m * m


def _pad_to(arr, n):
    if arr.size == n:
        return arr
    return jnp.concatenate([arr, jnp.zeros((n - arr.size,), arr.dtype)])


def _lstm_dec_body(audio_ref, Wf0, Uf0, bf0, Wb0, Ub0, bb0,
                   Wfs, Ufs, bfs, Wbs, Ubs, bbs, Wd, bd,
                   out_ref, gf_scr, gb_scr, hn_scr):
    def sigm(x):
        return 1.0 / (1.0 + jnp.exp(-x))

    def run_layer(h_val, Wf, Uf, bf, Wb, Ub, bb):
        gf_scr[...] = lax.dot_general(
            h_val, Wf, (((1,), (1,)), ((), ())),
            preferred_element_type=jnp.float32) + bf
        gb_scr[...] = lax.dot_general(
            h_val, Wb, (((1,), (1,)), ((), ())),
            preferred_element_type=jnp.float32) + bb

        def step(tt, carry, g_scr, U, col0):
            h, c = carry
            g = g_scr[pl.ds(tt, 1), :] + lax.dot_general(
                h, U, (((1,), (1,)), ((), ())),
                preferred_element_type=jnp.float32)
            i = sigm(g[:, 0:H])
            f = sigm(g[:, H:2 * H])
            gg = jnp.tanh(g[:, 2 * H:3 * H])
            o = sigm(g[:, 3 * H:4 * H])
            c = f * c + i * gg
            h = o * jnp.tanh(c)
            hn_scr[pl.ds(tt, 1), pl.ds(col0, H)] = h
            return (h, c)

        z = jnp.zeros((1, H), jnp.float32)
        lax.fori_loop(0, T, lambda t, cr: step(t, cr, gf_scr, Uf, 0), (z, z))
        lax.fori_loop(0, T, lambda t, cr: step(T - 1 - t, cr, gb_scr, Ub, H),
                      (z, z))
        return hn_scr[...]

    h = run_layer(audio_ref[...], Wf0[...], Uf0[...], bf0[...],
                  Wb0[...], Ub0[...], bb0[...])
    for i in range(4):
        h = run_layer(h, Wfs[i], Ufs[i], bfs[i], Wbs[i], Ubs[i], bbs[i])
    r = lax.dot_general(h, Wd[...], (((1,), (2,)), ((), ())),
                        preferred_element_type=jnp.float32)
    out_ref[...] = jnp.swapaxes(r, 0, 1) + bd[...]


def _lstm_dec(audio, l0, lrest, Wd, bd):
    return pl.pallas_call(
        _lstm_dec_body,
        out_shape=jax.ShapeDtypeStruct((40, T, 32), jnp.float32),
        scratch_shapes=[
            pltpu.VMEM((T, 4 * H), jnp.float32),
            pltpu.VMEM((T, 4 * H), jnp.float32),
            pltpu.VMEM((T, 2 * H), jnp.float32),
        ],
    )(audio, *l0, *lrest, Wd, bd)


def _pmm(y, wbd, bt, Np, D1, D2):
    Bn = 256
    NB = Np // Bn
    w_res = S * D1 * D2 * 4 <= 16 * 1024 * 1024

    def body(y_ref, w_ref, b_ref, out_ref):
        w = w_ref[pl.program_id(1)] if w_res else w_ref[0]
        r = lax.dot_general(y_ref[...], w, (((1,), (0,)), ((), ())),
                            preferred_element_type=jnp.float32)
        out_ref[...] = r + b_ref[...]

    if w_res:
        w_spec = pl.BlockSpec((S, D1, D2), lambda nb, s: (0, 0, 0))
    else:
        w_spec = pl.BlockSpec((1, D1, D2), lambda nb, s: (s, 0, 0))
    return pl.pallas_call(
        body,
        grid=(NB, S),
        in_specs=[
            pl.BlockSpec((Bn, D1), lambda nb, s: (nb, 0)),
            w_spec,
            pl.BlockSpec((1, D2), lambda nb, s: (0, 0)),
        ],
        out_specs=pl.BlockSpec((Bn, D2), lambda nb, s: (s * NB + nb, 0)),
        out_shape=jax.ShapeDtypeStruct((S * Np, D2), jnp.float32),
    )(y, wbd, bt)


def _make_pool(Np, D):
    bn = Np // NW
    ch = CH
    nch = bn // ch
    mesh = plsc.VectorSubcoreMesh(core_axis_name="c", subcore_axis_name="s")

    @functools.partial(
        pl.kernel, mesh=mesh,
        out_type=jax.ShapeDtypeStruct((Np, D), jnp.float32),
        scratch_types=[
            pltpu.VMEM((2, ch * 4), jnp.int32),
            pltpu.VMEM((2, ch * 4), jnp.float32),
            pltpu.VMEM((2, ch * 4, D), jnp.float32),
            pltpu.VMEM((ch, D), jnp.float32),
            pltpu.SemaphoreType.DMA,
            pltpu.SemaphoreType.DMA,
        ],
    )
    def pool_k(src_hbm, cols_hbm, vals_hbm, out_hbm, idx_v, w_v, g_v, y_v,
               sem0, sem1):
        wid = lax.axis_index("s") * NC + lax.axis_index("c")
        base = wid * bn
        sems = (sem0, sem1)

        def fetch(n0, b):
            pltpu.sync_copy(cols_hbm.at[pl.ds(n0 * 4, ch * 4)], idx_v.at[b])
            pltpu.sync_copy(vals_hbm.at[pl.ds(n0 * 4, ch * 4)], w_v.at[b])
            pltpu.async_copy(src_hbm.at[idx_v.at[b]], g_v.at[b], sems[b])

        def drain(b):
            pltpu.make_async_copy(src_hbm.at[idx_v.at[b]], g_v.at[b],
                                  sems[b]).wait()

        def compute(n0, b):
            wvecs = [w_v[b, pl.ds(16 * j, 16)] for j in range((ch * 4) // 16)]
            for i in range(ch):
                wv = wvecs[(4 * i) // 16]
                o = (4 * i) % 16
                w0 = wv[o]
                w1 = wv[o + 1]
                w2 = wv[o + 2]
                w3 = wv[o + 3]

                def vec(dv, c2):
                    sl = pl.ds(dv * 16, 16)
                    y_v[i, sl] = (g_v[b, 4 * i, sl] * w0
                                  + g_v[b, 4 * i + 1, sl] * w1
                                  + g_v[b, 4 * i + 2, sl] * w2
                                  + g_v[b, 4 * i + 3, sl] * w3)
                    return c2

                lax.fori_loop(0, D // 16, vec, 0, unroll=4)
            pltpu.sync_copy(y_v, out_hbm.at[pl.ds(n0, ch)])

        if nch % 2 == 0:
            fetch(base, 0)

            def pair(ci, carry):
                n0 = base + (2 * ci) * ch
                fetch(n0 + ch, 1)
                drain(0)
                compute(n0, 0)

                @pl.when(ci + 1 < nch // 2)
                def _():
                    fetch(n0 + 2 * ch, 0)

                drain(1)
                compute(n0 + ch, 1)
                return carry

            lax.fori_loop(0, nch // 2, pair, 0)
        else:
            def chunk(ci, carry):
                n0 = base + ci * ch
                fetch(n0, 0)
                drain(0)
                compute(n0, 0)
                return carry

            lax.fori_loop(0, nch, chunk, 0)

    return pool_k


def _make_gsum(Np, D, elu):
    bn = Np // NW
    ch = 4 if D > 512 else 8
    nch = bn // ch
    mesh = plsc.VectorSubcoreMesh(core_axis_name="c", subcore_axis_name="s")

    @functools.partial(
        pl.kernel, mesh=mesh,
        out_type=jax.ShapeDtypeStruct((Np, D), jnp.float32),
        scratch_types=[
            pltpu.VMEM((bn * S,), jnp.int32),
            pltpu.VMEM((2, ch * S, D), jnp.float32),
            pltpu.VMEM((2, ch, D), jnp.float32),
            pltpu.SemaphoreType.DMA,
            pltpu.SemaphoreType.DMA,
            pltpu.SemaphoreType.DMA,
            pltpu.SemaphoreType.DMA,
        ],
    )
    def gsum_k(p_hbm, gidx_hbm, out_hbm, idx_v, g_v, y_v,
               semg0, semg1, semo0, semo1):
        wid = lax.axis_index("s") * NC + lax.axis_index("c")
        base = wid * bn
        semg = (semg0, semg1)
        semo = (semo0, semo1)
        pltpu.sync_copy(gidx_hbm.at[pl.ds(base * S, bn * S)], idx_v)

        def gfetch(ci, b):
            pltpu.async_copy(
                p_hbm.at[idx_v.at[pl.ds(ci * ch * S, ch * S)]],
                g_v.at[b], semg[b])

        def gdrain(ci, b):
            pltpu.make_async_copy(
                p_hbm.at[idx_v.at[pl.ds(ci * ch * S, ch * S)]],
                g_v.at[b], semg[b]).wait()

        def owrite(n0, b):
            pltpu.async_copy(y_v.at[b], out_hbm.at[pl.ds(n0, ch)], semo[b])

        def odrain(b):
            pltpu.make_async_copy(y_v.at[b], out_hbm.at[pl.ds(base, ch)],
                                  semo[b]).wait()

        def compute(b):
            for i in range(ch):
                def vec(dv, c2):
                    sl = pl.ds(dv * 16, 16)
                    v = g_v[b, S * i, sl]
                    for s in range(1, S):
                        v = v + g_v[b, S * i + s, sl]
                    if elu:
                        v = jnp.where(v > 0.0, v,
                                      jnp.exp(jnp.minimum(v, 0.0)) - 1.0)
                    y_v[b, i, sl] = v
                    return c2

                lax.fori_loop(0, D // 16, vec, 0, unroll=4)

        if nch % 2 == 0:
            gfetch(0, 0)

            def pair(ci, carry):
                c0 = 2 * ci
                n0 = base + c0 * ch
                gfetch(c0 + 1, 1)
                gdrain(c0, 0)

                @pl.when(ci > 0)
                def _():
                    odrain(0)

                compute(0)
                owrite(n0, 0)

                @pl.when(ci + 1 < nch // 2)
                def _():
                    gfetch(c0 + 2, 0)

                gdrain(c0 + 1, 1)

                @pl.when(ci > 0)
                def _():
                    odrain(1)

                compute(1)
                owrite(n0 + ch, 1)
                return carry

            lax.fori_loop(0, nch // 2, pair, 0)
            odrain(0)
            odrain(1)
        else:
            def chunk(ci, carry):
                n0 = base + ci * ch
                gfetch(ci, 0)
                gdrain(ci, 0)
                compute(0)
                owrite(n0, 0)
                odrain(0)
                return carry

            lax.fori_loop(0, nch, chunk, 0)

    return gsum_k


def kernel(audio, actor, lstm_params, dec_W0, dec_b0, conv_params, up_vals,
           spiral_idx, up_rows, up_cols):
    (Wf0, Uf0, bf10, bf20, Wb0, Ub0, bb10, bb20) = lstm_params[0]
    l0 = (Wf0, Uf0, (bf10 + bf20).reshape(1, -1),
          Wb0, Ub0, (bb10 + bb20).reshape(1, -1))
    Wfs = jnp.stack([lstm_params[i][0] for i in range(1, 5)])
    Ufs = jnp.stack([lstm_params[i][1] for i in range(1, 5)])
    bfs = jnp.stack([(lstm_params[i][2] + lstm_params[i][3]).reshape(1, -1)
                     for i in range(1, 5)])
    Wbs = jnp.stack([lstm_params[i][4] for i in range(1, 5)])
    Ubs = jnp.stack([lstm_params[i][5] for i in range(1, 5)])
    bbs = jnp.stack([(lstm_params[i][6] + lstm_params[i][7]).reshape(1, -1)
                     for i in range(1, 5)])
    lrest = (Wfs, Ufs, bfs, Wbs, Ubs, bbs)
    Wd = dec_W0.reshape(40, 32, 2 * H)
    bd = dec_b0.reshape(40, 1, 32)

    x = _lstm_dec(audio, l0, lrest, Wd, bd)
    src = x.reshape(40, T * 32)
    C = 32
    Np = 40
    for k in range(5):
        if k < 4:
            lvl = 3 - k
            Nd = _LVL[lvl]
            Np = _pad_n(Nd)
            cols = _pad_to(up_cols[lvl], Np * 4)
            vals = _pad_to(up_vals[lvl], Np * 4)
            y = _make_pool(Np, T * C)(src, cols, vals)
            sidx = spiral_idx[lvl]
        else:
            y = src
            sidx = spiral_idx[0]
        W, b = conv_params[k]
        if k == 4:
            W = jnp.concatenate([W, jnp.zeros((1, W.shape[1]), W.dtype)])
            b = jnp.concatenate([b, jnp.zeros((1,), b.dtype)])
        Co = W.shape[0]
        wt = jnp.transpose(W.reshape(Co, S, C), (1, 2, 0))
        wbd = jnp.zeros((S, T * C, T * Co), jnp.float32)
        for t in range(T):
            wbd = lax.dynamic_update_slice(wbd, wt, (0, t * C, t * Co))
        bt = jnp.tile(b / S, (T,)).reshape(1, T * Co)
        P = _pmm(y, wbd, bt, Np, T * C, T * Co)
        gidx = sidx + (jnp.arange(S, dtype=jnp.int32) * Np)[None, :]
        gidx = _pad_to(gidx.reshape(-1), Np * S)
        src = _make_gsum(Np, T * Co, k < 4)(P, gidx)
        C = Co

    pred = src[:_LVL[0]].reshape(_LVL[0], T, 4)[:, :, :3].transpose(1, 0, 2)
    return pred + actor


# speedup vs baseline: 4.3285x; 1.0012x over previous
import functools

import jax
import jax.numpy as jnp
from jax import lax
from jax.experimental import pallas as pl
from jax.experimental.pallas import tpu as pltpu
from jax.experimental.pallas import tpu_sc as plsc

_LVL = [10000, 2500, 625, 157, 40]
S = 12
H = 64
T = 32
NC = 2
NS = 16
NW = NC * NS
CH = 8


def _pad_n(n):
    m = 8 * NW
    return (n + m - 1) // m * m


def _pad_to(arr, n):
    if arr.size == n:
        return arr
    return jnp.concatenate([arr, jnp.zeros((n - arr.size,), arr.dtype)])


def _lstm_dec_body(audio_ref, Wf0, Uf0, bf0, Wb0, Ub0, bb0,
                   Wfs, Ufs, bfs, Wbs, Ubs, bbs, Wd, bd,
                   out_ref, gf_scr, gb_scr, hn_scr):
    def sigm(x):
        return 1.0 / (1.0 + jnp.exp(-x))

    def run_layer(h_val, Wf, Uf, bf, Wb, Ub, bb):
        gf_scr[...] = lax.dot_general(
            h_val, Wf, (((1,), (1,)), ((), ())),
            preferred_element_type=jnp.float32) + bf
        gb_scr[...] = lax.dot_general(
            h_val, Wb, (((1,), (1,)), ((), ())),
            preferred_element_type=jnp.float32) + bb

        def step(tt, carry, g_scr, U, col0):
            h, c = carry
            g = g_scr[pl.ds(tt, 1), :] + lax.dot_general(
                h, U, (((1,), (1,)), ((), ())),
                preferred_element_type=jnp.float32)
            i = sigm(g[:, 0:H])
            f = sigm(g[:, H:2 * H])
            gg = jnp.tanh(g[:, 2 * H:3 * H])
            o = sigm(g[:, 3 * H:4 * H])
            c = f * c + i * gg
            h = o * jnp.tanh(c)
            hn_scr[pl.ds(tt, 1), pl.ds(col0, H)] = h
            return (h, c)

        z = jnp.zeros((1, H), jnp.float32)
        lax.fori_loop(0, T, lambda t, cr: step(t, cr, gf_scr, Uf, 0), (z, z))
        lax.fori_loop(0, T, lambda t, cr: step(T - 1 - t, cr, gb_scr, Ub, H),
                      (z, z))
        return hn_scr[...]

    h = run_layer(audio_ref[...], Wf0[...], Uf0[...], bf0[...],
                  Wb0[...], Ub0[...], bb0[...])
    for i in range(4):
        h = run_layer(h, Wfs[i], Ufs[i], bfs[i], Wbs[i], Ubs[i], bbs[i])
    r = lax.dot_general(h, Wd[...], (((1,), (2,)), ((), ())),
                        preferred_element_type=jnp.float32)
    out_ref[...] = jnp.swapaxes(r, 0, 1) + bd[...]


def _lstm_dec(audio, l0, lrest, Wd, bd):
    return pl.pallas_call(
        _lstm_dec_body,
        out_shape=jax.ShapeDtypeStruct((40, T, 32), jnp.float32),
        scratch_shapes=[
            pltpu.VMEM((T, 4 * H), jnp.float32),
            pltpu.VMEM((T, 4 * H), jnp.float32),
            pltpu.VMEM((T, 2 * H), jnp.float32),
        ],
    )(audio, *l0, *lrest, Wd, bd)


def _pmm(y, wbd, bt, Np, D1, D2):
    Bn = 256
    NB = Np // Bn
    w_res = S * D1 * D2 * 4 <= 16 * 1024 * 1024

    def body(y_ref, w_ref, b_ref, out_ref):
        w = w_ref[pl.program_id(1)] if w_res else w_ref[0]
        r = lax.dot_general(y_ref[...], w, (((1,), (0,)), ((), ())),
                            preferred_element_type=jnp.float32)
        out_ref[...] = r + b_ref[...]

    if w_res:
        w_spec = pl.BlockSpec((S, D1, D2), lambda nb, s: (0, 0, 0))
    else:
        w_spec = pl.BlockSpec((1, D1, D2), lambda nb, s: (s, 0, 0))
    return pl.pallas_call(
        body,
        grid=(NB, S),
        in_specs=[
            pl.BlockSpec((Bn, D1), lambda nb, s: (nb, 0)),
            w_spec,
            pl.BlockSpec((1, D2), lambda nb, s: (0, 0)),
        ],
        out_specs=pl.BlockSpec((Bn, D2), lambda nb, s: (s * NB + nb, 0)),
        out_shape=jax.ShapeDtypeStruct((S * Np, D2), jnp.float32),
    )(y, wbd, bt)


def _make_pool(Np, D):
    bn = Np // NW
    ch = CH
    nch = bn // ch
    mesh = plsc.VectorSubcoreMesh(core_axis_name="c", subcore_axis_name="s")

    @functools.partial(
        pl.kernel, mesh=mesh,
        out_type=jax.ShapeDtypeStruct((Np, D), jnp.float32),
        scratch_types=[
            pltpu.VMEM((2, ch * 4), jnp.int32),
            pltpu.VMEM((2, ch * 4), jnp.float32),
            pltpu.VMEM((2, ch * 4, D), jnp.float32),
            pltpu.VMEM((ch, D), jnp.float32),
            pltpu.SemaphoreType.DMA,
            pltpu.SemaphoreType.DMA,
        ],
    )
    def pool_k(src_hbm, cols_hbm, vals_hbm, out_hbm, idx_v, w_v, g_v, y_v,
               sem0, sem1):
        wid = lax.axis_index("s") * NC + lax.axis_index("c")
        base = wid * bn
        sems = (sem0, sem1)

        def fetch(n0, b):
            pltpu.sync_copy(cols_hbm.at[pl.ds(n0 * 4, ch * 4)], idx_v.at[b])
            pltpu.sync_copy(vals_hbm.at[pl.ds(n0 * 4, ch * 4)], w_v.at[b])
            pltpu.async_copy(src_hbm.at[idx_v.at[b]], g_v.at[b], sems[b])

        def drain(b):
            pltpu.make_async_copy(src_hbm.at[idx_v.at[b]], g_v.at[b],
                                  sems[b]).wait()

        def compute(n0, b):
            wvecs = [w_v[b, pl.ds(16 * j, 16)] for j in range((ch * 4) // 16)]
            for i in range(ch):
                wv = wvecs[(4 * i) // 16]
                o = (4 * i) % 16
                w0 = wv[o]
                w1 = wv[o + 1]
                w2 = wv[o + 2]
                w3 = wv[o + 3]

                def vec(dv, c2):
                    sl = pl.ds(dv * 16, 16)
                    y_v[i, sl] = (g_v[b, 4 * i, sl] * w0
                                  + g_v[b, 4 * i + 1, sl] * w1
                                  + g_v[b, 4 * i + 2, sl] * w2
                                  + g_v[b, 4 * i + 3, sl] * w3)
                    return c2

                lax.fori_loop(0, D // 16, vec, 0, unroll=4)
            pltpu.sync_copy(y_v, out_hbm.at[pl.ds(n0, ch)])

        if nch % 2 == 0:
            fetch(base, 0)

            def pair(ci, carry):
                n0 = base + (2 * ci) * ch
                fetch(n0 + ch, 1)
                drain(0)
                compute(n0, 0)

                @pl.when(ci + 1 < nch // 2)
                def _():
                    fetch(n0 + 2 * ch, 0)

                drain(1)
                compute(n0 + ch, 1)
                return carry

            lax.fori_loop(0, nch // 2, pair, 0)
        else:
            def chunk(ci, carry):
                n0 = base + ci * ch
                fetch(n0, 0)
                drain(0)
                compute(n0, 0)
                return carry

            lax.fori_loop(0, nch, chunk, 0)

    return pool_k


def _make_gsum(Np, D, elu):
    bn = Np // NW
    ch = 4 if D > 512 else 8
    nch = bn // ch
    mesh = plsc.VectorSubcoreMesh(core_axis_name="c", subcore_axis_name="s")

    @functools.partial(
        pl.kernel, mesh=mesh,
        out_type=jax.ShapeDtypeStruct((Np, D), jnp.float32),
        scratch_types=[
            pltpu.VMEM((bn * S,), jnp.int32),
            pltpu.VMEM((2, ch * S, D), jnp.float32),
            pltpu.VMEM((2, ch, D), jnp.float32),
            pltpu.SemaphoreType.DMA,
            pltpu.SemaphoreType.DMA,
            pltpu.SemaphoreType.DMA,
            pltpu.SemaphoreType.DMA,
        ],
    )
    def gsum_k(p_hbm, gidx_hbm, out_hbm, idx_v, g_v, y_v,
               semg0, semg1, semo0, semo1):
        wid = lax.axis_index("s") * NC + lax.axis_index("c")
        base = wid * bn
        semg = (semg0, semg1)
        semo = (semo0, semo1)
        pltpu.sync_copy(gidx_hbm.at[pl.ds(base * S, bn * S)], idx_v)

        hr = (ch * S) // 2

        def gfetch(ci, b):
            pltpu.async_copy(
                p_hbm.at[idx_v.at[pl.ds(ci * ch * S, hr)]],
                g_v.at[b, pl.ds(0, hr)], semg[b])
            pltpu.async_copy(
                p_hbm.at[idx_v.at[pl.ds(ci * ch * S + hr, hr)]],
                g_v.at[b, pl.ds(hr, hr)], semg[b])

        def gdrain(ci, b):
            pltpu.make_async_copy(
                p_hbm.at[idx_v.at[pl.ds(ci * ch * S, hr)]],
                g_v.at[b, pl.ds(0, hr)], semg[b]).wait()
            pltpu.make_async_copy(
                p_hbm.at[idx_v.at[pl.ds(ci * ch * S + hr, hr)]],
                g_v.at[b, pl.ds(hr, hr)], semg[b]).wait()

        def owrite(n0, b):
            pltpu.async_copy(y_v.at[b], out_hbm.at[pl.ds(n0, ch)], semo[b])

        def odrain(b):
            pltpu.make_async_copy(y_v.at[b], out_hbm.at[pl.ds(base, ch)],
                                  semo[b]).wait()

        def compute(b):
            for i in range(ch):
                def vec(dv, c2):
                    sl = pl.ds(dv * 16, 16)
                    v = g_v[b, S * i, sl]
                    for s in range(1, S):
                        v = v + g_v[b, S * i + s, sl]
                    if elu:
                        v = jnp.where(v > 0.0, v,
                                      jnp.exp(jnp.minimum(v, 0.0)) - 1.0)
                    y_v[b, i, sl] = v
                    return c2

                lax.fori_loop(0, D // 16, vec, 0, unroll=4)

        if nch % 2 == 0:
            gfetch(0, 0)

            def pair(ci, carry):
                c0 = 2 * ci
                n0 = base + c0 * ch
                gfetch(c0 + 1, 1)
                gdrain(c0, 0)

                @pl.when(ci > 0)
                def _():
                    odrain(0)

                compute(0)
                owrite(n0, 0)

                @pl.when(ci + 1 < nch // 2)
                def _():
                    gfetch(c0 + 2, 0)

                gdrain(c0 + 1, 1)

                @pl.when(ci > 0)
                def _():
                    odrain(1)

                compute(1)
                owrite(n0 + ch, 1)
                return carry

            lax.fori_loop(0, nch // 2, pair, 0)
            odrain(0)
            odrain(1)
        else:
            def chunk(ci, carry):
                n0 = base + ci * ch
                gfetch(ci, 0)
                gdrain(ci, 0)
                compute(0)
                owrite(n0, 0)
                odrain(0)
                return carry

            lax.fori_loop(0, nch, chunk, 0)

    return gsum_k


def kernel(audio, actor, lstm_params, dec_W0, dec_b0, conv_params, up_vals,
           spiral_idx, up_rows, up_cols):
    (Wf0, Uf0, bf10, bf20, Wb0, Ub0, bb10, bb20) = lstm_params[0]
    l0 = (Wf0, Uf0, (bf10 + bf20).reshape(1, -1),
          Wb0, Ub0, (bb10 + bb20).reshape(1, -1))
    Wfs = jnp.stack([lstm_params[i][0] for i in range(1, 5)])
    Ufs = jnp.stack([lstm_params[i][1] for i in range(1, 5)])
    bfs = jnp.stack([(lstm_params[i][2] + lstm_params[i][3]).reshape(1, -1)
                     for i in range(1, 5)])
    Wbs = jnp.stack([lstm_params[i][4] for i in range(1, 5)])
    Ubs = jnp.stack([lstm_params[i][5] for i in range(1, 5)])
    bbs = jnp.stack([(lstm_params[i][6] + lstm_params[i][7]).reshape(1, -1)
                     for i in range(1, 5)])
    lrest = (Wfs, Ufs, bfs, Wbs, Ubs, bbs)
    Wd = dec_W0.reshape(40, 32, 2 * H)
    bd = dec_b0.reshape(40, 1, 32)

    x = _lstm_dec(audio, l0, lrest, Wd, bd)
    src = x.reshape(40, T * 32)
    C = 32
    Np = 40
    for k in range(5):
        if k < 4:
            lvl = 3 - k
            Nd = _LVL[lvl]
            Np = _pad_n(Nd)
            cols = _pad_to(up_cols[lvl], Np * 4)
            vals = _pad_to(up_vals[lvl], Np * 4)
            y = _make_pool(Np, T * C)(src, cols, vals)
            sidx = spiral_idx[lvl]
        else:
            y = src
            sidx = spiral_idx[0]
        W, b = conv_params[k]
        if k == 4:
            W = jnp.concatenate([W, jnp.zeros((1, W.shape[1]), W.dtype)])
            b = jnp.concatenate([b, jnp.zeros((1,), b.dtype)])
        Co = W.shape[0]
        wt = jnp.transpose(W.reshape(Co, S, C), (1, 2, 0))
        wbd = jnp.zeros((S, T * C, T * Co), jnp.float32)
        for t in range(T):
            wbd = lax.dynamic_update_slice(wbd, wt, (0, t * C, t * Co))
        bt = jnp.tile(b / S, (T,)).reshape(1, T * Co)
        P = _pmm(y, wbd, bt, Np, T * C, T * Co)
        gidx = sidx + (jnp.arange(S, dtype=jnp.int32) * Np)[None, :]
        gidx = _pad_to(gidx.reshape(-1), Np * S)
        src = _make_gsum(Np, T * Co, k < 4)(P, gidx)
        C = Co

    pred = src[:_LVL[0]].reshape(_LVL[0], T, 4)[:, :, :3].transpose(1, 0, 2)
    return pred + actor
